# Initial kernel scaffold; baseline (speedup 1.0000x reference)
import jax
import jax.numpy as jnp
from jax.experimental import pallas as pl


def kernel(intent_embedding, scenario_ids, scenario_mask, table, W1, b1, W2, b2):
    raise NotImplementedError("write your pallas kernel here")


# speedup vs baseline: 82.4852x; 82.4852x over previous
import functools

import jax
import jax.numpy as jnp
from jax.experimental import pallas as pl
from jax.experimental.pallas import tpu as pltpu

B = 16384
W = 50
S = 19
D = 16
ED = 768
CD = 64
BLK = 1024


def _body(intent_ref, ids_ref, mask_ref, table_ref, W1_ref, b1_ref, W2_ref, b2_ref, out_ref):
    ids = ids_ref[...]
    mask = mask_ref[...]
    den_raw = jnp.sum(mask, axis=1, keepdims=True)
    den = jnp.where(den_raw > 0.0, den_raw, 1.0)

    acc = jnp.zeros((intent_ref.shape[0], CD), dtype=jnp.float32)
    for s in range(S):
        cnt = jnp.sum(mask * (ids == s).astype(jnp.float32), axis=1, keepdims=True)
        m_s = jnp.dot(table_ref[s:s + 1, :], W1_ref[ED + D * s: ED + D * (s + 1), :],
                      preferred_element_type=jnp.float32)
        acc = acc + cnt * m_s
    acc = acc / den

    h = jnp.dot(intent_ref[...], W1_ref[:ED, :], preferred_element_type=jnp.float32)
    h = jnp.maximum(h + acc + b1_ref[...], 0.0)
    out = jnp.dot(h, W2_ref[...], preferred_element_type=jnp.float32) + b2_ref[...]
    out_ref[...] = jnp.tanh(out)


@jax.jit
def kernel(intent_embedding, scenario_ids, scenario_mask, table, W1, b1, W2, b2):
    Bn = intent_embedding.shape[0]
    grid = (Bn // BLK,)
    return pl.pallas_call(
        _body,
        grid=grid,
        in_specs=[
            pl.BlockSpec((BLK, ED), lambda i: (i, 0)),
            pl.BlockSpec((BLK, W), lambda i: (i, 0)),
            pl.BlockSpec((BLK, W), lambda i: (i, 0)),
            pl.BlockSpec((S, D), lambda i: (0, 0)),
            pl.BlockSpec((ED + S * D, CD), lambda i: (0, 0)),
            pl.BlockSpec((1, CD), lambda i: (0, 0)),
            pl.BlockSpec((CD, 1), lambda i: (0, 0)),
            pl.BlockSpec((1, 1), lambda i: (0, 0)),
        ],
        out_specs=pl.BlockSpec((BLK, 1), lambda i: (i, 0)),
        out_shape=jax.ShapeDtypeStruct((Bn, 1), jnp.float32),
    )(intent_embedding, scenario_ids.astype(jnp.int32), scenario_mask,
      table, W1, b1.reshape(1, CD), W2, b2.reshape(1, 1))
